# initial kernel scaffold (unmeasured)
import jax
import jax.numpy as jnp
from jax import lax
from jax.experimental import pallas as pl
from jax.experimental.pallas import tpu as pltpu

N_DEV = 4


def kernel(x, w_mat, scale_x, scale_w):
    m_per, k = x.shape
    n_local = w_mat.shape[1]

    def body(x_ref, w_ref, sx_ref, sw_ref, out_ref, gather_ref,
             send_sems, recv_sems):
        my = lax.axis_index("i")
        left = lax.rem(my + N_DEV - 1, N_DEV)
        right = lax.rem(my + 1, N_DEV)

        barrier_sem = pltpu.get_barrier_semaphore()
        for nbr in (left, right):
            pl.semaphore_signal(barrier_sem, inc=1, device_id=(nbr,),
                                device_id_type=pl.DeviceIdType.MESH)
        pl.semaphore_wait(barrier_sem, 2)

        gather_ref[0, :, :] = x_ref[:, :]
        scale = sx_ref[0] * sw_ref[0]

        def compute(slot):
            origin = lax.rem(my - slot + N_DEV, N_DEV)
            acc = lax.dot_general(
                gather_ref[slot], w_ref[:, :],
                dimension_numbers=(((1,), (0,)), ((), ())),
                preferred_element_type=jnp.float32,
            )
            out_ref[pl.ds(origin * m_per, m_per), :] = jnp.maximum(
                acc * scale, 0.0)

        for h in range(N_DEV - 1):
            rdma = pltpu.make_async_remote_copy(
                src_ref=gather_ref.at[h],
                dst_ref=gather_ref.at[h + 1],
                send_sem=send_sems.at[h],
                recv_sem=recv_sems.at[h],
                device_id=(right,),
                device_id_type=pl.DeviceIdType.MESH,
            )
            rdma.start()
            compute(h)
            rdma.wait()
        compute(N_DEV - 1)

    return pl.pallas_call(
        body,
        out_shape=jax.ShapeDtypeStruct((N_DEV * m_per, n_local), jnp.float32),
        in_specs=[
            pl.BlockSpec(memory_space=pltpu.VMEM),
            pl.BlockSpec(memory_space=pltpu.VMEM),
            pl.BlockSpec(memory_space=pltpu.SMEM),
            pl.BlockSpec(memory_space=pltpu.SMEM),
        ],
        out_specs=pl.BlockSpec(memory_space=pltpu.VMEM),
        scratch_shapes=[
            pltpu.VMEM((N_DEV, m_per, k), x.dtype),
            pltpu.SemaphoreType.DMA((N_DEV - 1,)),
            pltpu.SemaphoreType.DMA((N_DEV - 1,)),
        ],
        compiler_params=pltpu.CompilerParams(collective_id=0),
    )(x, w_mat, scale_x, scale_w)


# baseline (device time: 199618 ns/iter reference)
import jax
import jax.numpy as jnp
from jax import lax
from jax.experimental import pallas as pl
from jax.experimental.pallas import tpu as pltpu

N_DEV = 4
FP8 = jnp.float8_e4m3fn


def kernel(x, w_mat, scale_x, scale_w):
    m_per, k = x.shape
    n_local = w_mat.shape[1]
    k_chunk = k // N_DEV

    def body(x_ref, w_ref, sx_ref, sw_ref, out_ref,
             x_stage, gather_ref, w8_ref, w_stage, out_stage,
             sem_x, sem_w, sem_out, send_sems, recv_sems):
        my = lax.axis_index("i")
        left = lax.rem(my + N_DEV - 1, N_DEV)
        right = lax.rem(my + 1, N_DEV)

        cp_x = pltpu.make_async_copy(x_ref, x_stage, sem_x)
        cp_x.start()
        for c in range(N_DEV):
            cp_w = pltpu.make_async_copy(
                w_ref.at[pl.ds(c * k_chunk, k_chunk)], w_stage, sem_w)
            cp_w.start()
            cp_w.wait()
            w8_ref[pl.ds(c * k_chunk, k_chunk), :] = (
                w_stage[:, :].astype(FP8))
        cp_x.wait()
        gather_ref[0, :, :] = x_stage[:, :].astype(FP8)

        barrier_sem = pltpu.get_barrier_semaphore()
        for nbr in (left, right):
            pl.semaphore_signal(barrier_sem, inc=1, device_id=(nbr,),
                                device_id_type=pl.DeviceIdType.MESH)
        pl.semaphore_wait(barrier_sem, 2)

        scale = sx_ref[0] * sw_ref[0]

        def compute(slot):
            origin = lax.rem(my - slot + N_DEV, N_DEV)
            acc = lax.dot_general(
                gather_ref[slot], w8_ref[:, :],
                dimension_numbers=(((1,), (0,)), ((), ())),
                preferred_element_type=jnp.float32,
            )
            out_stage[:, :] = jnp.maximum(acc * scale, 0.0)
            st = pltpu.make_async_copy(
                out_stage, out_ref.at[pl.ds(origin * m_per, m_per)], sem_out)
            st.start()
            st.wait()

        for h in range(N_DEV - 1):
            rdma = pltpu.make_async_remote_copy(
                src_ref=gather_ref.at[h],
                dst_ref=gather_ref.at[h + 1],
                send_sem=send_sems.at[h],
                recv_sem=recv_sems.at[h],
                device_id=(right,),
                device_id_type=pl.DeviceIdType.MESH,
            )
            rdma.start()
            compute(h)
            rdma.wait()
        compute(N_DEV - 1)

    return pl.pallas_call(
        body,
        out_shape=jax.ShapeDtypeStruct((N_DEV * m_per, n_local), jnp.float32),
        in_specs=[
            pl.BlockSpec(memory_space=pl.ANY),
            pl.BlockSpec(memory_space=pl.ANY),
            pl.BlockSpec(memory_space=pltpu.SMEM),
            pl.BlockSpec(memory_space=pltpu.SMEM),
        ],
        out_specs=pl.BlockSpec(memory_space=pl.ANY),
        scratch_shapes=[
            pltpu.VMEM((m_per, k), jnp.float32),
            pltpu.VMEM((N_DEV, m_per, k), FP8),
            pltpu.VMEM((k, n_local), FP8),
            pltpu.VMEM((k // N_DEV, n_local), jnp.float32),
            pltpu.VMEM((m_per, n_local), jnp.float32),
            pltpu.SemaphoreType.DMA,
            pltpu.SemaphoreType.DMA,
            pltpu.SemaphoreType.DMA,
            pltpu.SemaphoreType.DMA((N_DEV - 1,)),
            pltpu.SemaphoreType.DMA((N_DEV - 1,)),
        ],
        compiler_params=pltpu.CompilerParams(
            collective_id=0,
            vmem_limit_bytes=63 * 1024 * 1024,
        ),
    )(x, w_mat, scale_x, scale_w)


# device time: 122860 ns/iter; 1.6248x vs baseline; 1.6248x over previous
import jax
import jax.numpy as jnp
from jax import lax
from jax.experimental import pallas as pl
from jax.experimental.pallas import tpu as pltpu

N_DEV = 4
FP8 = jnp.float8_e4m3fn


def kernel(x, w_mat, scale_x, scale_w):
    m_per, k = x.shape
    m_half = m_per // 2
    n_local = w_mat.shape[1]
    k_chunk = k // N_DEV

    def body(x_ref, w_ref, sx_ref, sw_ref, out_ref,
             x_stage, gA, gB, w8_ref, w_stage, outA, outB,
             sem_x, sem_w, sem_oA, sem_oB,
             sendA, recvA, sendB, recvB):
        my = lax.axis_index("i")
        left = lax.rem(my + N_DEV - 1, N_DEV)
        right = lax.rem(my + 1, N_DEV)

        cp_x = pltpu.make_async_copy(x_ref, x_stage, sem_x)
        cp_x.start()
        cp_x.wait()
        gA[0, :, :] = x_stage[pl.ds(0, m_half), :].astype(FP8)
        gB[0, :, :] = x_stage[pl.ds(m_half, m_half), :].astype(FP8)

        barrier_sem = pltpu.get_barrier_semaphore()
        for nbr in (left, right):
            pl.semaphore_signal(barrier_sem, inc=1, device_id=(nbr,),
                                device_id_type=pl.DeviceIdType.MESH)
        pl.semaphore_wait(barrier_sem, 2)

        scale = sx_ref[0] * sw_ref[0]

        def stage_w():
            for c in range(N_DEV):
                cp_w = pltpu.make_async_copy(
                    w_ref.at[pl.ds(c * k_chunk, k_chunk)], w_stage, sem_w)
                cp_w.start()
                cp_w.wait()
                w8_ref[pl.ds(c * k_chunk, k_chunk), :] = (
                    w_stage[:, :].astype(FP8))

        def compute(slot, g, out_stage, sem_o, is_b):
            origin = lax.rem(my + (slot if is_b else -slot) + N_DEV, N_DEV)
            acc = lax.dot_general(
                g[slot], w8_ref[:, :],
                dimension_numbers=(((1,), (0,)), ((), ())),
                preferred_element_type=jnp.float32,
            )
            out_stage[:, :] = jnp.maximum(acc * scale, 0.0)
            row = origin * m_per + (m_half if is_b else 0)
            st = pltpu.make_async_copy(
                out_stage, out_ref.at[pl.ds(row, m_half)], sem_o)
            st.start()
            st.wait()

        for h in range(N_DEV - 1):
            rdmaA = pltpu.make_async_remote_copy(
                src_ref=gA.at[h], dst_ref=gA.at[h + 1],
                send_sem=sendA.at[h], recv_sem=recvA.at[h],
                device_id=(right,), device_id_type=pl.DeviceIdType.MESH,
            )
            rdmaB = pltpu.make_async_remote_copy(
                src_ref=gB.at[h], dst_ref=gB.at[h + 1],
                send_sem=sendB.at[h], recv_sem=recvB.at[h],
                device_id=(left,), device_id_type=pl.DeviceIdType.MESH,
            )
            rdmaA.start()
            rdmaB.start()
            if h == 0:
                stage_w()
            compute(h, gA, outA, sem_oA, is_b=False)
            compute(h, gB, outB, sem_oB, is_b=True)
            rdmaA.wait()
            rdmaB.wait()
        compute(N_DEV - 1, gA, outA, sem_oA, is_b=False)
        compute(N_DEV - 1, gB, outB, sem_oB, is_b=True)

    return pl.pallas_call(
        body,
        out_shape=jax.ShapeDtypeStruct((N_DEV * m_per, n_local), jnp.float32),
        in_specs=[
            pl.BlockSpec(memory_space=pl.ANY),
            pl.BlockSpec(memory_space=pl.ANY),
            pl.BlockSpec(memory_space=pltpu.SMEM),
            pl.BlockSpec(memory_space=pltpu.SMEM),
        ],
        out_specs=pl.BlockSpec(memory_space=pl.ANY),
        scratch_shapes=[
            pltpu.VMEM((m_per, k), jnp.float32),
            pltpu.VMEM((N_DEV, m_half, k), FP8),
            pltpu.VMEM((N_DEV, m_half, k), FP8),
            pltpu.VMEM((k, n_local), FP8),
            pltpu.VMEM((k_chunk, n_local), jnp.float32),
            pltpu.VMEM((m_half, n_local), jnp.float32),
            pltpu.VMEM((m_half, n_local), jnp.float32),
            pltpu.SemaphoreType.DMA,
            pltpu.SemaphoreType.DMA,
            pltpu.SemaphoreType.DMA,
            pltpu.SemaphoreType.DMA,
            pltpu.SemaphoreType.DMA((N_DEV - 1,)),
            pltpu.SemaphoreType.DMA((N_DEV - 1,)),
            pltpu.SemaphoreType.DMA((N_DEV - 1,)),
            pltpu.SemaphoreType.DMA((N_DEV - 1,)),
        ],
        compiler_params=pltpu.CompilerParams(
            collective_id=0,
            vmem_limit_bytes=63 * 1024 * 1024,
        ),
    )(x, w_mat, scale_x, scale_w)


# device time: 109362 ns/iter; 1.8253x vs baseline; 1.1234x over previous
import jax
import jax.numpy as jnp
from jax import lax
from jax.experimental import pallas as pl
from jax.experimental.pallas import tpu as pltpu

N_DEV = 4
S = 2
FP8 = jnp.float8_e4m3fn


def kernel(x, w_mat, scale_x, scale_w):
    m_per, k = x.shape
    m_half = m_per // 2
    sub = m_half // S
    n_local = w_mat.shape[1]
    k_chunk = k // N_DEV
    n_hops = N_DEV - 1

    def body(x_ref, w_ref, sx_ref, sw_ref, out_ref,
             x_stage, gA, gB, w8_ref, w_stage, outbuf,
             sems_q, sem_w, sems_o, sendA, recvA, sendB, recvB):
        my = lax.axis_index("i")
        left = lax.rem(my + N_DEV - 1, N_DEV)
        right = lax.rem(my + 1, N_DEV)

        cp_q = []
        for q in range(4):
            cp = pltpu.make_async_copy(
                x_ref.at[pl.ds(q * sub, sub)],
                x_stage.at[pl.ds(q * sub, sub)], sems_q.at[q])
            cp.start()
            cp_q.append(cp)

        barrier_sem = pltpu.get_barrier_semaphore()
        for nbr in (left, right):
            pl.semaphore_signal(barrier_sem, inc=1, device_id=(nbr,),
                                device_id_type=pl.DeviceIdType.MESH)
        pl.semaphore_wait(barrier_sem, 2)

        def mk(g, send_sems, recv_sems, h, s, nbr):
            return pltpu.make_async_remote_copy(
                src_ref=g.at[S * h + s],
                dst_ref=g.at[S * (h + 1) + s],
                send_sem=send_sems.at[S * h + s],
                recv_sem=recv_sems.at[S * h + s],
                device_id=(nbr,), device_id_type=pl.DeviceIdType.MESH,
            )
        rdmaA = [[mk(gA, sendA, recvA, h, s, right) for s in range(S)]
                 for h in range(n_hops)]
        rdmaB = [[mk(gB, sendB, recvB, h, s, left) for s in range(S)]
                 for h in range(n_hops)]

        for s in range(S):
            cp_q[s].wait()
            gA[S * 0 + s, :, :] = x_stage[pl.ds(s * sub, sub), :].astype(FP8)
            rdmaA[0][s].start()
            cp_q[2 + s].wait()
            gB[S * 0 + s, :, :] = (
                x_stage[pl.ds(m_half + s * sub, sub), :].astype(FP8))
            rdmaB[0][s].start()

        for c in range(N_DEV):
            cp_w = pltpu.make_async_copy(
                w_ref.at[pl.ds(c * k_chunk, k_chunk)], w_stage, sem_w)
            cp_w.start()
            cp_w.wait()
            w8_ref[pl.ds(c * k_chunk, k_chunk), :] = w_stage[:, :].astype(FP8)

        scale = sx_ref[0] * sw_ref[0]

        pending = [None, None]
        buf_idx = [0]

        def compute_sub(g, h, s, is_b):
            i = buf_idx[0]
            buf_idx[0] = 1 - i
            if pending[i] is not None:
                pending[i].wait()
            acc = lax.dot_general(
                g[S * h + s], w8_ref[:, :],
                dimension_numbers=(((1,), (0,)), ((), ())),
                preferred_element_type=jnp.float32,
            )
            outbuf[i, :, :] = jnp.maximum(acc * scale, 0.0)
            origin = lax.rem(my + (h if is_b else -h) + N_DEV, N_DEV)
            row = origin * m_per + (m_half if is_b else 0) + s * sub
            st = pltpu.make_async_copy(
                outbuf.at[i], out_ref.at[pl.ds(row, sub)], sems_o.at[i])
            st.start()
            pending[i] = st

        for s in range(S):
            compute_sub(gA, 0, s, is_b=False)
            compute_sub(gB, 0, s, is_b=True)

        for h in range(1, n_hops):
            for s in range(S):
                rdmaA[h - 1][s].wait_recv()
                rdmaA[h][s].start()
                rdmaB[h - 1][s].wait_recv()
                rdmaB[h][s].start()
            for s in range(S):
                compute_sub(gA, h, s, is_b=False)
                compute_sub(gB, h, s, is_b=True)

        for s in range(S):
            rdmaA[n_hops - 1][s].wait_recv()
            compute_sub(gA, n_hops, s, is_b=False)
            rdmaB[n_hops - 1][s].wait_recv()
            compute_sub(gB, n_hops, s, is_b=True)

        for h in range(n_hops):
            for s in range(S):
                rdmaA[h][s].wait_send()
                rdmaB[h][s].wait_send()
        for i in range(2):
            if pending[i] is not None:
                pending[i].wait()

    return pl.pallas_call(
        body,
        out_shape=jax.ShapeDtypeStruct((N_DEV * m_per, n_local), jnp.float32),
        in_specs=[
            pl.BlockSpec(memory_space=pl.ANY),
            pl.BlockSpec(memory_space=pl.ANY),
            pl.BlockSpec(memory_space=pltpu.SMEM),
            pl.BlockSpec(memory_space=pltpu.SMEM),
        ],
        out_specs=pl.BlockSpec(memory_space=pl.ANY),
        scratch_shapes=[
            pltpu.VMEM((m_per, k), jnp.float32),
            pltpu.VMEM((N_DEV * S, sub, k), FP8),
            pltpu.VMEM((N_DEV * S, sub, k), FP8),
            pltpu.VMEM((k, n_local), FP8),
            pltpu.VMEM((k_chunk, n_local), jnp.float32),
            pltpu.VMEM((2, sub, n_local), jnp.float32),
            pltpu.SemaphoreType.DMA((4,)),
            pltpu.SemaphoreType.DMA,
            pltpu.SemaphoreType.DMA((2,)),
            pltpu.SemaphoreType.DMA((n_hops * S,)),
            pltpu.SemaphoreType.DMA((n_hops * S,)),
            pltpu.SemaphoreType.DMA((n_hops * S,)),
            pltpu.SemaphoreType.DMA((n_hops * S,)),
        ],
        compiler_params=pltpu.CompilerParams(
            collective_id=0,
            vmem_limit_bytes=63 * 1024 * 1024,
        ),
    )(x, w_mat, scale_x, scale_w)


# device time: 108430 ns/iter; 1.8410x vs baseline; 1.0086x over previous
import jax
import jax.numpy as jnp
from jax import lax
from jax.experimental import pallas as pl
from jax.experimental.pallas import tpu as pltpu

N_DEV = 4
S = 4
FP8 = jnp.float8_e4m3fn


def kernel(x, w_mat, scale_x, scale_w):
    m_per, k = x.shape
    m_half = m_per // 2
    sub = m_half // S
    n_local = w_mat.shape[1]
    k_chunk = k // N_DEV
    n_hops = N_DEV - 1

    def body(x_ref, w_ref, sx_ref, sw_ref, out_ref,
             x_stage, gA, gB, w8_ref, w_stage, outbuf,
             sems_q, sem_w, sems_o, sendA, recvA, sendB, recvB):
        my = lax.axis_index("i")
        left = lax.rem(my + N_DEV - 1, N_DEV)
        right = lax.rem(my + 1, N_DEV)

        load_order = [q for pair in zip(range(S), range(S, 2 * S))
                      for q in pair]
        cp_q = {}

        def start_load(q):
            c = pltpu.make_async_copy(
                x_ref.at[pl.ds(q * sub, sub)],
                x_stage.at[pl.ds(q * sub, sub)], sems_q.at[q])
            c.start()
            cp_q[q] = c

        start_load(load_order[0])
        start_load(load_order[1])

        barrier_sem = pltpu.get_barrier_semaphore()
        for nbr in (left, right):
            pl.semaphore_signal(barrier_sem, inc=1, device_id=(nbr,),
                                device_id_type=pl.DeviceIdType.MESH)
        pl.semaphore_wait(barrier_sem, 2)

        def mk(g, send_sems, recv_sems, h, s, nbr):
            return pltpu.make_async_remote_copy(
                src_ref=g.at[S * h + s],
                dst_ref=g.at[S * (h + 1) + s],
                send_sem=send_sems.at[S * h + s],
                recv_sem=recv_sems.at[S * h + s],
                device_id=(nbr,), device_id_type=pl.DeviceIdType.MESH,
            )
        rdmaA = [[mk(gA, sendA, recvA, h, s, right) for s in range(S)]
                 for h in range(n_hops)]
        rdmaB = [[mk(gB, sendB, recvB, h, s, left) for s in range(S)]
                 for h in range(n_hops)]

        for idx, q in enumerate(load_order):
            if idx + 2 < len(load_order):
                start_load(load_order[idx + 2])
            cp_q[q].wait()
            if q < S:
                gA[q, :, :] = x_stage[pl.ds(q * sub, sub), :].astype(FP8)
                rdmaA[0][q].start()
            else:
                gB[q - S, :, :] = x_stage[pl.ds(q * sub, sub), :].astype(FP8)
                rdmaB[0][q - S].start()

        for c in range(N_DEV):
            cp_w = pltpu.make_async_copy(
                w_ref.at[pl.ds(c * k_chunk, k_chunk)], w_stage, sem_w)
            cp_w.start()
            cp_w.wait()
            w8_ref[pl.ds(c * k_chunk, k_chunk), :] = w_stage[:, :].astype(FP8)

        scale = sx_ref[0] * sw_ref[0]

        pending = [None, None]
        buf_idx = [0]

        def compute_sub(g, h, s, is_b):
            i = buf_idx[0]
            buf_idx[0] = 1 - i
            if pending[i] is not None:
                pending[i].wait()
            acc = lax.dot_general(
                g[S * h + s], w8_ref[:, :],
                dimension_numbers=(((1,), (0,)), ((), ())),
                preferred_element_type=jnp.float32,
            )
            outbuf[i, :, :] = jnp.maximum(acc * scale, 0.0)
            origin = lax.rem(my + (h if is_b else -h) + N_DEV, N_DEV)
            row = origin * m_per + (m_half if is_b else 0) + s * sub
            st = pltpu.make_async_copy(
                outbuf.at[i], out_ref.at[pl.ds(row, sub)], sems_o.at[i])
            st.start()
            pending[i] = st

        for s in range(S):
            compute_sub(gA, 0, s, is_b=False)
            compute_sub(gB, 0, s, is_b=True)

        for h in range(1, n_hops):
            for s in range(S):
                rdmaA[h - 1][s].wait_recv()
                rdmaA[h][s].start()
                rdmaB[h - 1][s].wait_recv()
                rdmaB[h][s].start()
            for s in range(S):
                compute_sub(gA, h, s, is_b=False)
                compute_sub(gB, h, s, is_b=True)

        for s in range(S):
            rdmaA[n_hops - 1][s].wait_recv()
            compute_sub(gA, n_hops, s, is_b=False)
            rdmaB[n_hops - 1][s].wait_recv()
            compute_sub(gB, n_hops, s, is_b=True)

        for h in range(n_hops):
            for s in range(S):
                rdmaA[h][s].wait_send()
                rdmaB[h][s].wait_send()
        for i in range(2):
            if pending[i] is not None:
                pending[i].wait()

    return pl.pallas_call(
        body,
        out_shape=jax.ShapeDtypeStruct((N_DEV * m_per, n_local), jnp.float32),
        in_specs=[
            pl.BlockSpec(memory_space=pl.ANY),
            pl.BlockSpec(memory_space=pl.ANY),
            pl.BlockSpec(memory_space=pltpu.SMEM),
            pl.BlockSpec(memory_space=pltpu.SMEM),
        ],
        out_specs=pl.BlockSpec(memory_space=pl.ANY),
        scratch_shapes=[
            pltpu.VMEM((m_per, k), jnp.float32),
            pltpu.VMEM((N_DEV * S, sub, k), FP8),
            pltpu.VMEM((N_DEV * S, sub, k), FP8),
            pltpu.VMEM((k, n_local), FP8),
            pltpu.VMEM((k_chunk, n_local), jnp.float32),
            pltpu.VMEM((2, sub, n_local), jnp.float32),
            pltpu.SemaphoreType.DMA((2 * S,)),
            pltpu.SemaphoreType.DMA,
            pltpu.SemaphoreType.DMA((2,)),
            pltpu.SemaphoreType.DMA((n_hops * S,)),
            pltpu.SemaphoreType.DMA((n_hops * S,)),
            pltpu.SemaphoreType.DMA((n_hops * S,)),
            pltpu.SemaphoreType.DMA((n_hops * S,)),
        ],
        compiler_params=pltpu.CompilerParams(
            collective_id=0,
            vmem_limit_bytes=63 * 1024 * 1024,
        ),
    )(x, w_mat, scale_x, scale_w)
